# baseline (speedup 1.0000x reference)
import jax
import jax.numpy as jnp
from jax import lax
from jax.experimental import pallas as pl
from jax.experimental.pallas import tpu as pltpu
from jax.experimental.pallas import tpu_sc as plsc

B = 16384
F = 12
D_MODEL = 768
FACTOR_DIM = 64
KPAD = 16

BLK = 2048
NBLK = B // BLK
SC_BLOCKS = 1
TC_BLOCKS = NBLK - SC_BLOCKS
SC_ROWS = SC_BLOCKS * BLK
SC_ROW0 = TC_BLOCKS * BLK

NC, NS = 2, 16
NW = NC * NS
ROWS_W = SC_ROWS // NW
GROUPS = ROWS_W // 16


def _sc_body(idx_hbm, val_hbm, out_hbm, idx_v, val_v, wmat_v, sem):
    wid = lax.axis_index("s") * NC + lax.axis_index("c")
    c0 = SC_ROW0 + (wid // 2) * 128
    half = (wid % 2) * ROWS_W
    ci = pltpu.async_copy(idx_hbm.at[:, pl.ds(c0, 128)], idx_v, sem)
    cv = pltpu.async_copy(val_hbm.at[:, pl.ds(c0, 128)], val_v, sem)

    zeros = jnp.zeros((16,), jnp.float32)

    def zero_body(i, _):
        for j in range(8):
            wmat_v[i * 8 + j, :] = zeros
        return 0

    lax.fori_loop(0, ROWS_W // 8, zero_body, 0)

    ci.wait()
    cv.wait()

    lanes = lax.iota(jnp.int32, 16)

    def group_body(g, _):
        lb = g * 16 + lanes

        def f_body(f, _):
            iv = idx_v[f, pl.ds(half + g * 16, 16)]
            vv = val_v[f, pl.ds(half + g * 16, 16)]
            vv = jnp.minimum(jnp.maximum(vv, 0.0), 1.0)
            plsc.addupdate_scatter(wmat_v, [lb, iv], vv)
            return 0

        lax.fori_loop(0, F, f_body, 0)
        return 0

    lax.fori_loop(0, GROUPS, group_body, 0)

    pltpu.sync_copy(wmat_v, out_hbm.at[pl.ds(wid * ROWS_W, ROWS_W), :])


_sc_wmat = pl.kernel(
    _sc_body,
    out_type=jax.ShapeDtypeStruct((SC_ROWS, KPAD), jnp.float32),
    mesh=plsc.VectorSubcoreMesh(core_axis_name="c", subcore_axis_name="s"),
    compiler_params=pltpu.CompilerParams(needs_layout_passes=False),
    scratch_types=[
        pltpu.VMEM((F, 128), jnp.int32),
        pltpu.VMEM((F, 128), jnp.float32),
        pltpu.VMEM((ROWS_W, KPAD), jnp.float32),
        pltpu.SemaphoreType.DMA,
    ],
)


def _matmul_m(ftp_ref, wp_ref):
    return jnp.dot(ftp_ref[...], wp_ref[...],
                   preferred_element_type=jnp.float32)


def _proja_body(idxT_ref, valT_ref, ftp_ref, wp_ref, b_ref, out_ref):
    idxT = idxT_ref[...]
    vT = jnp.clip(valT_ref[...], 0.0, 1.0)
    rows = []
    for k in range(F):
        rows.append(jnp.sum(jnp.where(idxT == k, vT, 0.0),
                            axis=0, keepdims=True))
    rows.append(jnp.zeros((KPAD - F, idxT.shape[1]), jnp.float32))
    wmat_t = jnp.concatenate(rows, axis=0)
    m = _matmul_m(ftp_ref, wp_ref)
    out = lax.dot_general(wmat_t, m, (((0,), (0,)), ((), ())),
                          preferred_element_type=jnp.float32)
    out_ref[...] = out + b_ref[...]


BLKA = 2048


def _proja(idxT, valT, ft_pad, W_proj, b2d):
    return pl.pallas_call(
        _proja_body,
        grid=(SC_ROW0 // BLKA,),
        in_specs=[
            pl.BlockSpec((F, BLKA), lambda i: (0, i)),
            pl.BlockSpec((F, BLKA), lambda i: (0, i)),
            pl.BlockSpec((KPAD, FACTOR_DIM), lambda i: (0, 0)),
            pl.BlockSpec((FACTOR_DIM, D_MODEL), lambda i: (0, 0)),
            pl.BlockSpec((1, D_MODEL), lambda i: (0, 0)),
        ],
        out_specs=pl.BlockSpec((BLKA, D_MODEL), lambda i: (i, 0)),
        out_shape=jax.ShapeDtypeStruct((B, D_MODEL), jnp.float32),
    )(idxT, valT, ft_pad, W_proj, b2d)


def _projb_body(w_ref, ftp_ref, wp_ref, b_ref, outa_ref, out_ref):
    m = _matmul_m(ftp_ref, wp_ref)
    out_ref[...] = jnp.dot(w_ref[...], m,
                           preferred_element_type=jnp.float32) + b_ref[...]


BLKB = 512


def _projb(wmat_sc, ft_pad, W_proj, b2d, outa):
    nb0 = SC_ROW0 // BLKB
    return pl.pallas_call(
        _projb_body,
        grid=(SC_ROWS // BLKB,),
        in_specs=[
            pl.BlockSpec((BLKB, KPAD), lambda i: (i, 0)),
            pl.BlockSpec((KPAD, FACTOR_DIM), lambda i: (0, 0)),
            pl.BlockSpec((FACTOR_DIM, D_MODEL), lambda i: (0, 0)),
            pl.BlockSpec((1, D_MODEL), lambda i: (0, 0)),
            pl.BlockSpec(memory_space=pl.ANY),
        ],
        out_specs=pl.BlockSpec((BLKB, D_MODEL),
                               lambda i: (i + nb0, 0)),
        out_shape=jax.ShapeDtypeStruct((B, D_MODEL), jnp.float32),
        input_output_aliases={4: 0},
    )(wmat_sc, ft_pad, W_proj, b2d, outa)


@jax.jit
def _run(indices, values, factor_table, W_proj, b_proj):
    idxT = indices.T
    valT = values.T
    wmat_sc = _sc_wmat(idxT, valT)
    ft_pad = jnp.pad(factor_table, ((0, KPAD - F), (0, 0)))
    b2d = b_proj.reshape(1, D_MODEL)
    outa = _proja(idxT, valT, ft_pad, W_proj, b2d)
    return _projb(wmat_sc, ft_pad, W_proj, b2d, outa)


def kernel(indices, values, factor_table, W_proj, b_proj):
    return _run(indices, values, factor_table, W_proj, b_proj)


# speedup vs baseline: 1.0133x; 1.0133x over previous
import jax
import jax.numpy as jnp
from jax import lax
from jax.experimental import pallas as pl
from jax.experimental.pallas import tpu as pltpu
from jax.experimental.pallas import tpu_sc as plsc

B = 16384
F = 12
D_MODEL = 768
FACTOR_DIM = 64
KPAD = 16

BLK = 2048
NBLK = B // BLK
SC_BLOCKS = 1
TC_BLOCKS = NBLK - SC_BLOCKS
SC_ROWS = SC_BLOCKS * BLK
SC_ROW0 = TC_BLOCKS * BLK

NC, NS = 2, 16
NW = NC * NS
ROWS_W = SC_ROWS // NW
GROUPS = ROWS_W // 16


def _sc_body(idx_hbm, val_hbm, out_hbm, idx_v, val_v, wmat_v, sem):
    wid = lax.axis_index("s") * NC + lax.axis_index("c")
    c0 = SC_ROW0 + (wid // 2) * 128
    half = (wid % 2) * ROWS_W
    ci = pltpu.async_copy(idx_hbm.at[:, pl.ds(c0, 128)], idx_v, sem)
    cv = pltpu.async_copy(val_hbm.at[:, pl.ds(c0, 128)], val_v, sem)

    zeros = jnp.zeros((16,), jnp.float32)

    def zero_body(i, _):
        for j in range(8):
            wmat_v[i * 8 + j, :] = zeros
        return 0

    lax.fori_loop(0, ROWS_W // 8, zero_body, 0)

    ci.wait()
    cv.wait()

    lanes = lax.iota(jnp.int32, 16)

    def group_body(g, _):
        lb = g * 16 + lanes

        def f_body(f, _):
            iv = idx_v[f, pl.ds(half + g * 16, 16)]
            vv = val_v[f, pl.ds(half + g * 16, 16)]
            vv = jnp.minimum(jnp.maximum(vv, 0.0), 1.0)
            plsc.addupdate_scatter(wmat_v, [lb, iv], vv)
            return 0

        lax.fori_loop(0, F, f_body, 0)
        return 0

    lax.fori_loop(0, GROUPS, group_body, 0)

    pltpu.sync_copy(wmat_v, out_hbm.at[pl.ds(wid * ROWS_W, ROWS_W), :])


_sc_wmat = pl.kernel(
    _sc_body,
    out_type=jax.ShapeDtypeStruct((SC_ROWS, KPAD), jnp.float32),
    mesh=plsc.VectorSubcoreMesh(core_axis_name="c", subcore_axis_name="s"),
    compiler_params=pltpu.CompilerParams(needs_layout_passes=False),
    scratch_types=[
        pltpu.VMEM((F, 128), jnp.int32),
        pltpu.VMEM((F, 128), jnp.float32),
        pltpu.VMEM((ROWS_W, KPAD), jnp.float32),
        pltpu.SemaphoreType.DMA,
    ],
)


def _matmul_m(ftp_ref, wp_ref):
    return jnp.dot(ftp_ref[...], wp_ref[...],
                   preferred_element_type=jnp.float32)


def _proja_body(idxT_ref, valT_ref, ftp_ref, wp_ref, b_ref, out_ref):
    idxT = idxT_ref[...]
    vT = jnp.clip(valT_ref[...], 0.0, 1.0)
    rows = []
    for k in range(F):
        rows.append(jnp.sum(jnp.where(idxT == k, vT, 0.0),
                            axis=0, keepdims=True))
    rows.append(jnp.zeros((KPAD - F, idxT.shape[1]), jnp.float32))
    wmat_t = jnp.concatenate(rows, axis=0)
    m = _matmul_m(ftp_ref, wp_ref)
    out = lax.dot_general(wmat_t, m, (((0,), (0,)), ((), ())),
                          preferred_element_type=jnp.float32)
    out_ref[...] = out + b_ref[...]


BLKA = 2048


def _proja(idxT, valT, ft_pad, W_proj, b2d):
    return pl.pallas_call(
        _proja_body,
        grid=(SC_ROW0 // BLKA,),
        in_specs=[
            pl.BlockSpec((F, BLKA), lambda i: (0, i)),
            pl.BlockSpec((F, BLKA), lambda i: (0, i)),
            pl.BlockSpec((KPAD, FACTOR_DIM), lambda i: (0, 0)),
            pl.BlockSpec((FACTOR_DIM, D_MODEL), lambda i: (0, 0)),
            pl.BlockSpec((1, D_MODEL), lambda i: (0, 0)),
        ],
        out_specs=pl.BlockSpec((BLKA, D_MODEL), lambda i: (i, 0)),
        out_shape=jax.ShapeDtypeStruct((B, D_MODEL), jnp.float32),
    )(idxT, valT, ft_pad, W_proj, b2d)


def _projb_body(w_ref, ftp_ref, wp_ref, b_ref, outa_ref, out_ref):
    m = _matmul_m(ftp_ref, wp_ref)
    out_ref[...] = jnp.dot(w_ref[...], m,
                           preferred_element_type=jnp.float32) + b_ref[...]


BLKB = 2048


def _projb(wmat_sc, ft_pad, W_proj, b2d, outa):
    nb0 = SC_ROW0 // BLKB
    return pl.pallas_call(
        _projb_body,
        grid=(SC_ROWS // BLKB,),
        in_specs=[
            pl.BlockSpec((BLKB, KPAD), lambda i: (i, 0)),
            pl.BlockSpec((KPAD, FACTOR_DIM), lambda i: (0, 0)),
            pl.BlockSpec((FACTOR_DIM, D_MODEL), lambda i: (0, 0)),
            pl.BlockSpec((1, D_MODEL), lambda i: (0, 0)),
            pl.BlockSpec(memory_space=pl.ANY),
        ],
        out_specs=pl.BlockSpec((BLKB, D_MODEL),
                               lambda i: (i + nb0, 0)),
        out_shape=jax.ShapeDtypeStruct((B, D_MODEL), jnp.float32),
        input_output_aliases={4: 0},
    )(wmat_sc, ft_pad, W_proj, b2d, outa)


@jax.jit
def _run(indices, values, factor_table, W_proj, b_proj):
    idxT = indices.T
    valT = values.T
    wmat_sc = _sc_wmat(idxT, valT)
    ft_pad = jnp.pad(factor_table, ((0, KPAD - F), (0, 0)))
    b2d = b_proj.reshape(1, D_MODEL)
    outa = _proja(idxT, valT, ft_pad, W_proj, b2d)
    return _projb(wmat_sc, ft_pad, W_proj, b2d, outa)


def kernel(indices, values, factor_table, W_proj, b_proj):
    return _run(indices, values, factor_table, W_proj, b_proj)
